# baseline (speedup 1.0000x reference)
import jax
import jax.numpy as jnp
from jax.experimental import pallas as pl
from jax.experimental.pallas import tpu as pltpu

_NK = 4


def _sage_body(a_ref, x_ref, wl_ref, bl_ref, wr_ref, o_ref, num_s, cnt_s):
    k = pl.program_id(0)
    bk = a_ref.shape[0]
    a = a_ref[...]
    x_blk = x_ref[pl.ds(k * bk, bk), :]
    dn = (((0,), (0,)), ((), ()))
    num_p = jax.lax.dot_general(a, x_blk, dn, preferred_element_type=jnp.float32)
    ones = jnp.ones((bk, 1), dtype=jnp.float32)
    cnt_p = jax.lax.dot_general(a, ones, dn, preferred_element_type=jnp.float32)
    dt = (((1,), (1,)), ((), ()))

    @pl.when(k == 0)
    def _init():
        num_s[...] = num_p
        cnt_s[...] = cnt_p
        xf = x_ref[...]
        xh = xf.astype(jnp.bfloat16)
        xl = (xf - xh.astype(jnp.float32)).astype(jnp.bfloat16)
        wr = wr_ref[...]
        wrh = wr.astype(jnp.bfloat16)
        wrl = (wr - wrh.astype(jnp.float32)).astype(jnp.bfloat16)
        root = (jax.lax.dot_general(xh, wrh, dt, preferred_element_type=jnp.float32)
                + jax.lax.dot_general(xh, wrl, dt, preferred_element_type=jnp.float32)
                + jax.lax.dot_general(xl, wrh, dt, preferred_element_type=jnp.float32))
        o_ref[...] = bl_ref[...] + root

    @pl.when(jnp.logical_and(k > 0, k < _NK - 1))
    def _accum():
        num_s[...] += num_p
        cnt_s[...] += cnt_p

    @pl.when(k == _NK - 1)
    def _epilogue():
        agg = (num_s[...] + num_p) / jnp.maximum(cnt_s[...] + cnt_p, 1.0)
        o_ref[...] += jax.lax.dot_general(
            agg, wl_ref[...], dt, preferred_element_type=jnp.float32)


def kernel(features, adjacency_matrix, W_l, b_l, W_r):
    n, d = features.shape
    bk = n // _NK
    return pl.pallas_call(
        _sage_body,
        grid=(_NK,),
        in_specs=[
            pl.BlockSpec((bk, n), lambda k: (k, 0)),
            pl.BlockSpec((n, d), lambda k: (0, 0)),
            pl.BlockSpec((d, d), lambda k: (0, 0)),
            pl.BlockSpec((1, d), lambda k: (0, 0)),
            pl.BlockSpec((d, d), lambda k: (0, 0)),
        ],
        out_specs=pl.BlockSpec((n, d), lambda k: (0, 0)),
        out_shape=jax.ShapeDtypeStruct((n, d), jnp.float32),
        scratch_shapes=[
            pltpu.VMEM((n, d), jnp.float32),
            pltpu.VMEM((n, 1), jnp.float32),
        ],
        compiler_params=pltpu.CompilerParams(
            dimension_semantics=("arbitrary",)),
    )(adjacency_matrix, features, W_l, b_l.reshape(1, d), W_r)


# speedup vs baseline: 1.1946x; 1.1946x over previous
import jax
import jax.numpy as jnp
from jax.experimental import pallas as pl
from jax.experimental.pallas import tpu as pltpu

_NK = 2


def _sage_body(a_ref, x_ref, wl_ref, bl_ref, wr_ref, o_ref, num_s, cnt_s):
    k = pl.program_id(0)
    bk = a_ref.shape[0]
    a = a_ref[...].astype(jnp.bfloat16)
    x_blk = x_ref[pl.ds(k * bk, bk), :].astype(jnp.bfloat16)
    dn = (((0,), (0,)), ((), ()))
    num_p = jax.lax.dot_general(a, x_blk, dn, preferred_element_type=jnp.float32)
    ones = jnp.ones((bk, 1), dtype=jnp.bfloat16)
    cnt_p = jax.lax.dot_general(a, ones, dn, preferred_element_type=jnp.float32)
    dt = (((1,), (1,)), ((), ()))

    @pl.when(k == 0)
    def _init():
        num_s[...] = num_p
        cnt_s[...] = cnt_p
        xf = x_ref[...]
        xh = xf.astype(jnp.bfloat16)
        xl = (xf - xh.astype(jnp.float32)).astype(jnp.bfloat16)
        wr = wr_ref[...]
        wrh = wr.astype(jnp.bfloat16)
        wrl = (wr - wrh.astype(jnp.float32)).astype(jnp.bfloat16)
        root = (jax.lax.dot_general(xh, wrh, dt, preferred_element_type=jnp.float32)
                + jax.lax.dot_general(xh, wrl, dt, preferred_element_type=jnp.float32)
                + jax.lax.dot_general(xl, wrh, dt, preferred_element_type=jnp.float32))
        o_ref[...] = bl_ref[...] + root

    @pl.when(k == _NK - 1)
    def _epilogue():
        agg = ((num_s[...] + num_p)
               / jnp.maximum(cnt_s[...] + cnt_p, 1.0)).astype(jnp.bfloat16)
        wl = wl_ref[...].astype(jnp.bfloat16)
        o_ref[...] += jax.lax.dot_general(
            agg, wl, dt, preferred_element_type=jnp.float32)


def kernel(features, adjacency_matrix, W_l, b_l, W_r):
    n, d = features.shape
    bk = n // _NK
    return pl.pallas_call(
        _sage_body,
        grid=(_NK,),
        in_specs=[
            pl.BlockSpec((bk, n), lambda k: (k, 0)),
            pl.BlockSpec((n, d), lambda k: (0, 0)),
            pl.BlockSpec((d, d), lambda k: (0, 0)),
            pl.BlockSpec((1, d), lambda k: (0, 0)),
            pl.BlockSpec((d, d), lambda k: (0, 0)),
        ],
        out_specs=pl.BlockSpec((n, d), lambda k: (0, 0)),
        out_shape=jax.ShapeDtypeStruct((n, d), jnp.float32),
        scratch_shapes=[
            pltpu.VMEM((n, d), jnp.float32),
            pltpu.VMEM((n, 1), jnp.float32),
        ],
        compiler_params=pltpu.CompilerParams(
            dimension_semantics=("arbitrary",)),
    )(adjacency_matrix, features, W_l, b_l.reshape(1, d), W_r)
